# baseline (speedup 1.0000x reference)
import functools

import jax
import jax.numpy as jnp
from jax import lax
from jax.experimental import pallas as pl
from jax.experimental.pallas import tpu as pltpu
from jax.experimental.pallas import tpu_sc as plsc

_NC = 2
_NS = 16
_NW = _NC * _NS

_K = 128
_P = 2
_NBUF = 3


def _gather_kernel(idx_hbm, table_hbm, out_hbm, idx_v, rows_v, gsem, osem,
                   *, n_super):
    n = n_super
    wid = lax.axis_index("s") * _NC + lax.axis_index("c")
    super0 = wid * n

    pltpu.sync_copy(idx_hbm.at[wid], idx_v)

    def g_pairs(j, s):
        return [
            (table_hbm.at[idx_v.at[j * _P + p]], rows_v.at[s].at[p])
            for p in range(_P)
        ]

    def start_gather(j, s):
        for src, dst in g_pairs(j, s):
            pltpu.async_copy(src, dst, gsem[s])

    def wait_gather(j, s):
        for src, dst in g_pairs(j, s):
            pltpu.make_async_copy(src, dst, gsem[s]).wait()

    def out_ref(j):
        return out_hbm.at[pl.ds((super0 + j) * _P, _P)]

    def start_out(j, s):
        pltpu.async_copy(rows_v.at[s], out_ref(j), osem[s])

    def wait_out(j, s):
        pltpu.make_async_copy(rows_v.at[s], out_ref(j), osem[s]).wait()

    assert n >= 4 and (n - 1) % _NBUF == 0
    start_gather(0, 0)

    def step(j, s, *, drain, issue):
        s_next = (s + 1) % _NBUF
        if drain:
            wait_out(j - 2, s_next)
        if issue:
            start_gather(j + 1, s_next)
        wait_gather(j, s)
        start_out(j, s)

    step(0, 0, drain=False, issue=True)
    step(1, 1, drain=False, issue=True)
    step(2, 2, drain=True, issue=True)

    @pl.loop(3, n - 1, step=_NBUF)
    def _(c):
        for b in range(_NBUF):
            j = c + b
            step(j, b, drain=True, issue=True)

    jl = n - 1
    sl = jl % _NBUF
    wait_gather(jl, sl)
    start_out(jl, sl)
    for j in range(n - 3, n):
        wait_out(j, j % _NBUF)


def kernel(token_ids, weight):
    b, t = token_ids.shape
    d = weight.shape[1]
    flat_n = b * t
    assert flat_n % (_NW * _K * _P) == 0
    n_super = flat_n // (_NW * _K * _P)

    idx3d = token_ids.reshape(_NW, n_super * _P, _K).astype(jnp.int32)

    grid_kernel = functools.partial(_gather_kernel, n_super=n_super)
    mesh = plsc.VectorSubcoreMesh(core_axis_name="c", subcore_axis_name="s")
    out = pl.kernel(
        grid_kernel,
        out_type=jax.ShapeDtypeStruct((flat_n // _K, _K, d), jnp.float32),
        mesh=mesh,
        scratch_types=[
            pltpu.VMEM((n_super * _P, _K), jnp.int32),
            pltpu.VMEM((_NBUF, _P, _K, d), jnp.float32),
            [pltpu.SemaphoreType.DMA] * _NBUF,
            [pltpu.SemaphoreType.DMA] * _NBUF,
        ],
    )(idx3d, weight)
    return out.reshape(b, t, d)


# speedup vs baseline: 1.0179x; 1.0179x over previous
import functools

import jax
import jax.numpy as jnp
from jax import lax
from jax.experimental import pallas as pl
from jax.experimental.pallas import tpu as pltpu
from jax.experimental.pallas import tpu_sc as plsc

_NC = 2
_NS = 16
_NW = _NC * _NS

_K = 128
_NBUF = 6
_A = 3


def _gather_kernel(idx_hbm, table_hbm, out_hbm, idx_v, rows_v, gsem, osem,
                   *, chunks_per_worker):
    n = chunks_per_worker
    wid = lax.axis_index("s") * _NC + lax.axis_index("c")
    chunk0 = wid * n

    pltpu.sync_copy(idx_hbm.at[wid], idx_v)

    def start_gather(j, s):
        pltpu.async_copy(table_hbm.at[idx_v.at[j]], rows_v.at[s], gsem[s])

    def wait_gather(j, s):
        pltpu.make_async_copy(
            table_hbm.at[idx_v.at[j]], rows_v.at[s], gsem[s]
        ).wait()

    def out_ref(j):
        return out_hbm.at[pl.ds((chunk0 + j) * _K, _K)]

    def start_out(j, s):
        pltpu.async_copy(rows_v.at[s], out_ref(j), osem[s])

    def wait_out(j, s):
        pltpu.make_async_copy(rows_v.at[s], out_ref(j), osem[s]).wait()

    for j in range(_A):
        start_gather(j, j)

    head = _NBUF - _A + ((n - _A) - (_NBUF - _A)) % _NBUF
    assert head >= _NBUF - _A and (n - _A - head) % _NBUF == 0

    for j in range(head):
        s_issue = (j + _A) % _NBUF
        if j + _A - _NBUF >= 0:
            wait_out(j + _A - _NBUF, s_issue)
        start_gather(j + _A, s_issue)
        wait_gather(j, j % _NBUF)
        start_out(j, j % _NBUF)

    r = head % _NBUF

    @pl.loop(head, n - _A, step=_NBUF)
    def _(c):
        for b in range(_NBUF):
            j = c + b
            s = (r + b) % _NBUF
            s_issue = (r + b + _A) % _NBUF
            wait_out(j + _A - _NBUF, s_issue)
            start_gather(j + _A, s_issue)
            wait_gather(j, s)
            start_out(j, s)

    for j in range(n - _A, n):
        s = j % _NBUF
        wait_gather(j, s)
        start_out(j, s)
    for j in range(n - _NBUF, n):
        wait_out(j, j % _NBUF)


def kernel(token_ids, weight):
    b, t = token_ids.shape
    d = weight.shape[1]
    flat_n = b * t
    assert flat_n % (_NW * _K) == 0
    chunks_per_worker = flat_n // (_NW * _K)

    idx3d = token_ids.reshape(_NW, chunks_per_worker, _K).astype(jnp.int32)

    grid_kernel = functools.partial(_gather_kernel,
                                    chunks_per_worker=chunks_per_worker)
    mesh = plsc.VectorSubcoreMesh(core_axis_name="c", subcore_axis_name="s")
    out = pl.kernel(
        grid_kernel,
        out_type=jax.ShapeDtypeStruct((flat_n, d), jnp.float32),
        mesh=mesh,
        scratch_types=[
            pltpu.VMEM((chunks_per_worker, _K), jnp.int32),
            pltpu.VMEM((_NBUF, _K, d), jnp.float32),
            [pltpu.SemaphoreType.DMA] * _NBUF,
            [pltpu.SemaphoreType.DMA] * _NBUF,
        ],
    )(idx3d, weight)
    return out.reshape(b, t, d)


# speedup vs baseline: 1.0253x; 1.0073x over previous
import functools

import jax
import jax.numpy as jnp
from jax import lax
from jax.experimental import pallas as pl
from jax.experimental.pallas import tpu as pltpu
from jax.experimental.pallas import tpu_sc as plsc

_NC = 2
_NS = 16
_NW = _NC * _NS

_K = 128
_NBUF = 7
_A = 4


def _gather_kernel(idx_hbm, table_hbm, out_hbm, idx_v, rows_v, gsem, osem,
                   *, chunks_per_worker):
    n = chunks_per_worker
    wid = lax.axis_index("s") * _NC + lax.axis_index("c")
    chunk0 = wid * n

    pltpu.sync_copy(idx_hbm.at[wid], idx_v)

    def start_gather(j, s):
        pltpu.async_copy(table_hbm.at[idx_v.at[j]], rows_v.at[s], gsem[s])

    def wait_gather(j, s):
        pltpu.make_async_copy(
            table_hbm.at[idx_v.at[j]], rows_v.at[s], gsem[s]
        ).wait()

    def out_ref(j):
        return out_hbm.at[pl.ds((chunk0 + j) * _K, _K)]

    def start_out(j, s):
        pltpu.async_copy(rows_v.at[s], out_ref(j), osem[s])

    def wait_out(j, s):
        pltpu.make_async_copy(rows_v.at[s], out_ref(j), osem[s]).wait()

    for j in range(_A):
        start_gather(j, j)

    head = _NBUF - _A + ((n - _A) - (_NBUF - _A)) % _NBUF
    assert head >= _NBUF - _A and (n - _A - head) % _NBUF == 0

    for j in range(head):
        s_issue = (j + _A) % _NBUF
        if j + _A - _NBUF >= 0:
            wait_out(j + _A - _NBUF, s_issue)
        start_gather(j + _A, s_issue)
        wait_gather(j, j % _NBUF)
        start_out(j, j % _NBUF)

    r = head % _NBUF

    @pl.loop(head, n - _A, step=_NBUF)
    def _(c):
        for b in range(_NBUF):
            j = c + b
            s = (r + b) % _NBUF
            s_issue = (r + b + _A) % _NBUF
            wait_out(j + _A - _NBUF, s_issue)
            start_gather(j + _A, s_issue)
            wait_gather(j, s)
            start_out(j, s)

    for j in range(n - _A, n):
        s = j % _NBUF
        wait_gather(j, s)
        start_out(j, s)
    for j in range(n - _NBUF, n):
        wait_out(j, j % _NBUF)


def kernel(token_ids, weight):
    b, t = token_ids.shape
    d = weight.shape[1]
    flat_n = b * t
    assert flat_n % (_NW * _K) == 0
    chunks_per_worker = flat_n // (_NW * _K)

    idx3d = token_ids.reshape(_NW, chunks_per_worker, _K).astype(jnp.int32)

    grid_kernel = functools.partial(_gather_kernel,
                                    chunks_per_worker=chunks_per_worker)
    mesh = plsc.VectorSubcoreMesh(core_axis_name="c", subcore_axis_name="s")
    out = pl.kernel(
        grid_kernel,
        out_type=jax.ShapeDtypeStruct((flat_n, d), jnp.float32),
        mesh=mesh,
        scratch_types=[
            pltpu.VMEM((chunks_per_worker, _K), jnp.int32),
            pltpu.VMEM((_NBUF, _K, d), jnp.float32),
            [pltpu.SemaphoreType.DMA] * _NBUF,
            [pltpu.SemaphoreType.DMA] * _NBUF,
        ],
    )(idx3d, weight)
    return out.reshape(b, t, d)
